# Initial kernel scaffold; baseline (speedup 1.0000x reference)
import jax
import jax.numpy as jnp
from jax.experimental import pallas as pl


def kernel(x, edge_index, W1, b1, W2, b2, a):
    raise NotImplementedError("write your pallas kernel here")


# speedup vs baseline: 8.1200x; 8.1200x over previous
import functools

import jax
import jax.numpy as jnp
from jax import lax
from jax.experimental import pallas as pl
from jax.experimental.pallas import tpu as pltpu
from jax.experimental.pallas import tpu_sc as plsc

NC = 2
NS = 16
CH = 128


def _mesh():
    return plsc.VectorSubcoreMesh(core_axis_name="c", subcore_axis_name="s")


def _make_deg(NP, nch):
    ZR = NP // NS

    @functools.partial(
        pl.kernel,
        out_type=jax.ShapeDtypeStruct((NC * NP,), jnp.float32),
        mesh=_mesh(),
        scratch_types=[
            pltpu.VMEM((nch, 1, CH), jnp.int32),
            pltpu.VMEM((CH,), jnp.float32),
            pltpu.VMEM((ZR,), jnp.float32),
            pltpu.VMEM_SHARED((NP,), jnp.float32),
        ],
    )
    def deg_kernel(dst_hbm, out_hbm, dst_v, ones_v, zer_v, deg_sh):
        cid = lax.axis_index("c")
        sid = lax.axis_index("s")
        wid = sid * NC + cid
        for k in range(CH // 16):
            ones_v[pl.ds(k * 16, 16)] = jnp.ones((16,), jnp.float32)
        for k in range(ZR // 16):
            zer_v[pl.ds(k * 16, 16)] = jnp.zeros((16,), jnp.float32)

        pltpu.sync_copy(zer_v, deg_sh.at[pl.ds(sid * ZR, ZR)])
        pltpu.sync_copy(dst_hbm.at[wid], dst_v)
        plsc.subcore_barrier()

        def chunk(c, carry):
            pltpu.sync_copy(ones_v, deg_sh.at[dst_v.at[c, 0]], add=True)
            return carry

        lax.fori_loop(0, nch, chunk, 0)
        plsc.subcore_barrier()
        pltpu.sync_copy(deg_sh.at[pl.ds(sid * ZR, ZR)],
                        out_hbm.at[pl.ds(cid * NP + sid * ZR, ZR)])

    return deg_kernel


def _make_scatter(NP, F, nch, col_split):
    ZR = NP // NS
    assert nch % 2 == 0 and ZR % CH == 0

    @functools.partial(
        pl.kernel,
        out_type=jax.ShapeDtypeStruct((NC, NP, F), jnp.float32),
        mesh=_mesh(),
        scratch_types=[
            pltpu.VMEM((2, 1, CH), jnp.int32),
            pltpu.VMEM((2, 1, CH), jnp.int32),
            pltpu.VMEM((2, CH, F), jnp.float32),
            pltpu.VMEM_SHARED((NP, F), jnp.float32),
            pltpu.SemaphoreType.DMA,
            pltpu.SemaphoreType.DMA,
        ],
    )
    def scat_kernel(src_hbm, dst_hbm, g_hbm, out_hbm,
                    src_v, dst_v, rows_v, acc_sh, sem0, sem1):
        cid = lax.axis_index("c")
        sid = lax.axis_index("s")
        sems = (sem0, sem1)

        def zrow(r, carry):
            for k in range(F // 16):
                rows_v[0, r, pl.ds(k * 16, 16)] = jnp.zeros((16,), jnp.float32)
            return carry

        lax.fori_loop(0, CH, zrow, 0)
        for z in range(ZR // CH):
            pltpu.sync_copy(rows_v.at[0],
                            acc_sh.at[pl.ds(sid * ZR + z * CH, CH)])
        plsc.subcore_barrier()

        def chunk2(c, carry):
            cps = []
            for b in range(2):
                if col_split:
                    pltpu.sync_copy(src_hbm.at[cid, sid, c + b], src_v.at[b])
                    pltpu.sync_copy(dst_hbm.at[sid, c + b], dst_v.at[b])
                else:
                    wid = sid * NC + cid
                    pltpu.sync_copy(src_hbm.at[wid, c + b], src_v.at[b])
                    pltpu.sync_copy(dst_hbm.at[wid, c + b], dst_v.at[b])
                cps.append(pltpu.async_copy(
                    g_hbm.at[src_v.at[b, 0]], rows_v.at[b], sems[b]))
            for b in range(2):
                cps[b].wait()
                pltpu.sync_copy(rows_v.at[b],
                                acc_sh.at[dst_v.at[b, 0]], add=True)
            return carry

        lax.fori_loop(0, nch // 2, lambda i, c: chunk2(i * 2, c), 0)
        plsc.subcore_barrier()
        pltpu.sync_copy(acc_sh.at[pl.ds(sid * ZR, ZR)],
                        out_hbm.at[cid, pl.ds(sid * ZR, ZR)])

    return scat_kernel


def _prelu_tc(t, a):
    return jnp.where(t >= 0, t, a * t)


def _tc_a_body(x_ref, w_ref, degp_ref, g_ref, dinv_ref):
    deg = degp_ref[:, 0] + degp_ref[:, 1] + 1.0
    dinv = lax.rsqrt(deg)[:, None]
    h = jnp.dot(x_ref[...], w_ref[...], preferred_element_type=jnp.float32)
    g = h * dinv
    F = h.shape[1] // 2
    g_ref[0] = g[:, :F]
    g_ref[1] = g[:, F:]
    dinv_ref[...] = dinv


def _tc_d_body(s1_ref, g1_ref, dinv_ref, b1_ref, w2_ref, a_ref, g2_ref):
    a = a_ref[0, 0]
    dinv = dinv_ref[...]
    F = s1_ref.shape[2]
    t0 = _prelu_tc(dinv * (s1_ref[0] + g1_ref[0]) + b1_ref[0, :F], a)
    t1 = _prelu_tc(dinv * (s1_ref[1] + g1_ref[1]) + b1_ref[0, F:], a)
    h2 = (jnp.dot(t0, w2_ref[:F], preferred_element_type=jnp.float32)
          + jnp.dot(t1, w2_ref[F:], preferred_element_type=jnp.float32))
    g2_ref[...] = h2 * dinv


def _tc_f_body(s2_ref, g2_ref, dinv_ref, b2_ref, a_ref, out_ref):
    a = a_ref[0, 0]
    dinv = dinv_ref[...]
    t = dinv * (s2_ref[0] + s2_ref[1] + g2_ref[...]) + b2_ref[0, :]
    out_ref[...] = _prelu_tc(t, a)


def kernel(x, edge_index, W1, b1, W2, b2, a):
    N, IN_FT = x.shape
    HID = W1.shape[1]
    OUT_FT = W2.shape[1]
    E = edge_index.shape[1]
    F1 = HID // 2
    NW = NC * NS

    NP = ((N + NS * 64 - 1) // (NS * 64)) * (NS * 64)
    if NP == N:
        NP += NS * 64
    EP = ((E + NW * CH * 2 - 1) // (NW * CH * 2)) * (NW * CH * 2)
    nch1 = EP // (NS * CH)
    nch2 = EP // (NW * CH)

    src = edge_index[0]
    dst = edge_index[1]
    pad = EP - E
    src_p = jnp.concatenate([src, jnp.zeros((pad,), jnp.int32)])
    dst_p = jnp.concatenate([dst, jnp.full((pad,), N, jnp.int32)])
    src16 = src_p.reshape(NS, nch1, 1, CH)
    src16b = jnp.stack([src16, src16 + N])
    dst16 = dst_p.reshape(NS, nch1, 1, CH)
    src32 = src_p.reshape(NW, nch2, 1, CH)
    dst32 = dst_p.reshape(NW, nch2, 1, CH)

    degp = _make_deg(NP, nch2)(dst32).reshape(NC, NP)

    BR = 1000
    assert N % BR == 0
    grid = (N // BR,)
    g1, dinv = pl.pallas_call(
        _tc_a_body,
        grid=grid,
        in_specs=[
            pl.BlockSpec((BR, IN_FT), lambda i: (i, 0)),
            pl.BlockSpec((IN_FT, HID), lambda i: (0, 0)),
            pl.BlockSpec((BR, NC), lambda i: (i, 0)),
        ],
        out_specs=[
            pl.BlockSpec((NC, BR, F1), lambda i: (0, i, 0)),
            pl.BlockSpec((BR, 1), lambda i: (i, 0)),
        ],
        out_shape=[
            jax.ShapeDtypeStruct((NC, N, F1), jnp.float32),
            jax.ShapeDtypeStruct((N, 1), jnp.float32),
        ],
    )(x, W1, degp[:, :N].T)

    g1_tab = g1.reshape(NC * N, F1)
    s1 = _make_scatter(NP, F1, nch1, True)(src16b, dst16, g1_tab)

    a2 = a.reshape(1, 1)
    g2 = pl.pallas_call(
        _tc_d_body,
        grid=grid,
        in_specs=[
            pl.BlockSpec((NC, BR, F1), lambda i: (0, i, 0)),
            pl.BlockSpec((NC, BR, F1), lambda i: (0, i, 0)),
            pl.BlockSpec((BR, 1), lambda i: (i, 0)),
            pl.BlockSpec((1, HID), lambda i: (0, 0)),
            pl.BlockSpec((HID, OUT_FT), lambda i: (0, 0)),
            pl.BlockSpec((1, 1), lambda i: (0, 0)),
        ],
        out_specs=pl.BlockSpec((BR, OUT_FT), lambda i: (i, 0)),
        out_shape=jax.ShapeDtypeStruct((N, OUT_FT), jnp.float32),
    )(s1[:, :N, :], g1, dinv, b1.reshape(1, HID), W2, a2)

    s2 = _make_scatter(NP, OUT_FT, nch2, False)(src32, dst32, g2)

    out = pl.pallas_call(
        _tc_f_body,
        grid=grid,
        in_specs=[
            pl.BlockSpec((NC, BR, OUT_FT), lambda i: (0, i, 0)),
            pl.BlockSpec((BR, OUT_FT), lambda i: (i, 0)),
            pl.BlockSpec((BR, 1), lambda i: (i, 0)),
            pl.BlockSpec((1, OUT_FT), lambda i: (0, 0)),
            pl.BlockSpec((1, 1), lambda i: (0, 0)),
        ],
        out_specs=pl.BlockSpec((BR, OUT_FT), lambda i: (i, 0)),
        out_shape=jax.ShapeDtypeStruct((N, OUT_FT), jnp.float32),
    )(s2[:, :N, :], g2, dinv, b2.reshape(1, OUT_FT), a2)
    return out


# speedup vs baseline: 15.2546x; 1.8786x over previous
import functools

import jax
import jax.numpy as jnp
from jax import lax
from jax.experimental import pallas as pl
from jax.experimental.pallas import tpu as pltpu
from jax.experimental.pallas import tpu_sc as plsc

NC = 2
NS = 16
CH = 112
NB = 3


def _mesh():
    return plsc.VectorSubcoreMesh(core_axis_name="c", subcore_axis_name="s")


def _make_deg(NP, nch):
    ZR = NP // NS

    @functools.partial(
        pl.kernel,
        out_type=jax.ShapeDtypeStruct((NC * NP,), jnp.float32),
        mesh=_mesh(),
        scratch_types=[
            pltpu.VMEM((nch, 1, CH), jnp.int32),
            pltpu.VMEM((CH,), jnp.float32),
            pltpu.VMEM((ZR,), jnp.float32),
            pltpu.VMEM_SHARED((NP,), jnp.float32),
        ],
    )
    def deg_kernel(dst_hbm, out_hbm, dst_v, ones_v, zer_v, deg_sh):
        cid = lax.axis_index("c")
        sid = lax.axis_index("s")
        wid = sid * NC + cid
        for k in range(CH // 16):
            ones_v[pl.ds(k * 16, 16)] = jnp.ones((16,), jnp.float32)
        for k in range(ZR // 16):
            zer_v[pl.ds(k * 16, 16)] = jnp.zeros((16,), jnp.float32)

        pltpu.sync_copy(zer_v, deg_sh.at[pl.ds(sid * ZR, ZR)])
        pltpu.sync_copy(dst_hbm.at[wid], dst_v)
        plsc.subcore_barrier()

        def chunk(c, carry):
            pltpu.sync_copy(ones_v, deg_sh.at[dst_v.at[c, 0]], add=True)
            return carry

        lax.fori_loop(0, nch, chunk, 0)
        plsc.subcore_barrier()
        pltpu.sync_copy(deg_sh.at[pl.ds(sid * ZR, ZR)],
                        out_hbm.at[pl.ds(cid * NP + sid * ZR, ZR)])

    return deg_kernel


def _make_scatter(NP, F, nch, col_split):
    ZR = NP // NS
    assert nch % NB == 0 and ZR % 64 == 0

    @functools.partial(
        pl.kernel,
        out_type=jax.ShapeDtypeStruct((NC, NP, F), jnp.float32),
        mesh=_mesh(),
        scratch_types=[
            pltpu.VMEM((NB, 1, CH), jnp.int32),
            pltpu.VMEM((NB, 1, CH), jnp.int32),
            pltpu.VMEM((NB, CH, F), jnp.float32),
            pltpu.VMEM_SHARED((NP, F), jnp.float32),
            [pltpu.SemaphoreType.DMA] * NB,
            [pltpu.SemaphoreType.DMA] * NB,
        ],
    )
    def scat_kernel(src_hbm, dst_hbm, g_hbm, out_hbm,
                    src_v, dst_v, rows_v, acc_sh, sems_g, sems_s):
        cid = lax.axis_index("c")
        sid = lax.axis_index("s")

        def load_idx(cc, b):
            if col_split:
                pltpu.sync_copy(src_hbm.at[cid, sid, cc], src_v.at[b])
                pltpu.sync_copy(dst_hbm.at[sid, cc], dst_v.at[b])
            else:
                wid = sid * NC + cid
                pltpu.sync_copy(src_hbm.at[wid, cc], src_v.at[b])
                pltpu.sync_copy(dst_hbm.at[wid, cc], dst_v.at[b])

        def fire_gather(b):
            pltpu.async_copy(g_hbm.at[src_v.at[b, 0]], rows_v.at[b],
                             sems_g[b])

        def wait_gather(b):
            pltpu.make_async_copy(g_hbm.at[src_v.at[b, 0]], rows_v.at[b],
                                  sems_g[b]).wait()

        def fire_scatter(b):
            pltpu.async_copy(rows_v.at[b], acc_sh.at[dst_v.at[b, 0]],
                             sems_s[b], add=True)

        def wait_scatter(b):
            pltpu.make_async_copy(rows_v.at[b], acc_sh.at[dst_v.at[b, 0]],
                                  sems_s[b]).wait()

        def zrow(r, carry):
            for k in range(F // 16):
                rows_v[0, r, pl.ds(k * 16, 16)] = jnp.zeros((16,), jnp.float32)
            return carry

        lax.fori_loop(0, 64, zrow, 0)
        for z in range(ZR // 64):
            pltpu.sync_copy(rows_v.at[0, pl.ds(0, 64)],
                            acc_sh.at[pl.ds(sid * ZR + z * 64, 64)])

        NP_dummy = NP - 8
        for b in range(NB):
            for k in range(CH // 16):
                dst_v[b, 0, pl.ds(k * 16, 16)] = jnp.full(
                    (16,), NP_dummy, jnp.int32)
        for b in range(NB):
            fire_scatter(b)
        wait_scatter(0)
        load_idx(0, 0)
        fire_gather(0)
        plsc.subcore_barrier()

        def visit(cc, b, bnxt, last):
            if not last:
                wait_scatter(bnxt)
                load_idx(cc + 1, bnxt)
                fire_gather(bnxt)
            wait_gather(b)
            fire_scatter(b)

        def group(i, carry):
            for b in range(NB):
                cc = i * NB + b
                bnxt = (b + 1) % NB

                @pl.when(cc + 1 < nch)
                def _():
                    visit(cc, b, bnxt, False)

                @pl.when(cc + 1 >= nch)
                def _():
                    wait_gather(b)
                    fire_scatter(b)
            return carry

        lax.fori_loop(0, nch // NB, group, 0)
        for b in range(NB):
            wait_scatter(b)
        plsc.subcore_barrier()
        pltpu.sync_copy(acc_sh.at[pl.ds(sid * ZR, ZR)],
                        out_hbm.at[cid, pl.ds(sid * ZR, ZR)])

    return scat_kernel


def _prelu_tc(t, a):
    return jnp.where(t >= 0, t, a * t)


def _tc_a_body(x_ref, w_ref, degp_ref, g_ref, dinv_ref):
    deg = degp_ref[:, 0] + degp_ref[:, 1] + 1.0
    dinv = lax.rsqrt(deg)[:, None]
    h = jnp.dot(x_ref[...], w_ref[...], preferred_element_type=jnp.float32)
    g = h * dinv
    F = h.shape[1] // 2
    g_ref[0] = g[:, :F]
    g_ref[1] = g[:, F:]
    dinv_ref[...] = dinv


def _tc_d_body(s1_ref, g1_ref, dinv_ref, b1_ref, w2_ref, a_ref, g2_ref):
    a = a_ref[0, 0]
    dinv = dinv_ref[...]
    F = s1_ref.shape[2]
    t0 = _prelu_tc(dinv * (s1_ref[0] + g1_ref[0]) + b1_ref[0, :F], a)
    t1 = _prelu_tc(dinv * (s1_ref[1] + g1_ref[1]) + b1_ref[0, F:], a)
    h2 = (jnp.dot(t0, w2_ref[:F], preferred_element_type=jnp.float32)
          + jnp.dot(t1, w2_ref[F:], preferred_element_type=jnp.float32))
    g2_ref[...] = h2 * dinv


def _tc_f_body(s2_ref, g2_ref, dinv_ref, b2_ref, a_ref, out_ref):
    a = a_ref[0, 0]
    dinv = dinv_ref[...]
    t = dinv * (s2_ref[0] + s2_ref[1] + g2_ref[...]) + b2_ref[0, :]
    out_ref[...] = _prelu_tc(t, a)


def kernel(x, edge_index, W1, b1, W2, b2, a):
    N, IN_FT = x.shape
    HID = W1.shape[1]
    OUT_FT = W2.shape[1]
    E = edge_index.shape[1]
    F1 = HID // 2
    NW = NC * NS

    NP = ((N + NS * 64 - 1) // (NS * 64)) * (NS * 64)
    if NP == N:
        NP += NS * 64
    EP = ((E + NW * CH * NB - 1) // (NW * CH * NB)) * (NW * CH * NB)
    nch1 = EP // (NS * CH)
    nch2 = EP // (NW * CH)

    src = edge_index[0]
    dst = edge_index[1]
    pad = EP - E
    src_p = jnp.concatenate([src, jnp.zeros((pad,), jnp.int32)])
    dst_p = jnp.concatenate([dst, jnp.full((pad,), N, jnp.int32)])
    src16 = src_p.reshape(NS, nch1, 1, CH)
    src16b = jnp.stack([src16, src16 + N])
    dst16 = dst_p.reshape(NS, nch1, 1, CH)
    src32 = src_p.reshape(NW, nch2, 1, CH)
    dst32 = dst_p.reshape(NW, nch2, 1, CH)

    degp = _make_deg(NP, nch2)(dst32).reshape(NC, NP)

    BR = 1000
    assert N % BR == 0
    grid = (N // BR,)
    g1, dinv = pl.pallas_call(
        _tc_a_body,
        grid=grid,
        in_specs=[
            pl.BlockSpec((BR, IN_FT), lambda i: (i, 0)),
            pl.BlockSpec((IN_FT, HID), lambda i: (0, 0)),
            pl.BlockSpec((BR, NC), lambda i: (i, 0)),
        ],
        out_specs=[
            pl.BlockSpec((NC, BR, F1), lambda i: (0, i, 0)),
            pl.BlockSpec((BR, 1), lambda i: (i, 0)),
        ],
        out_shape=[
            jax.ShapeDtypeStruct((NC, N, F1), jnp.float32),
            jax.ShapeDtypeStruct((N, 1), jnp.float32),
        ],
    )(x, W1, degp[:, :N].T)

    g1_tab = g1.reshape(NC * N, F1)
    s1 = _make_scatter(NP, F1, nch1, True)(src16b, dst16, g1_tab)

    a2 = a.reshape(1, 1)
    g2 = pl.pallas_call(
        _tc_d_body,
        grid=grid,
        in_specs=[
            pl.BlockSpec((NC, BR, F1), lambda i: (0, i, 0)),
            pl.BlockSpec((NC, BR, F1), lambda i: (0, i, 0)),
            pl.BlockSpec((BR, 1), lambda i: (i, 0)),
            pl.BlockSpec((1, HID), lambda i: (0, 0)),
            pl.BlockSpec((HID, OUT_FT), lambda i: (0, 0)),
            pl.BlockSpec((1, 1), lambda i: (0, 0)),
        ],
        out_specs=pl.BlockSpec((BR, OUT_FT), lambda i: (i, 0)),
        out_shape=jax.ShapeDtypeStruct((N, OUT_FT), jnp.float32),
    )(s1[:, :N, :], g1, dinv, b1.reshape(1, HID), W2, a2)

    s2 = _make_scatter(NP, OUT_FT, nch2, False)(src32, dst32, g2)

    out = pl.pallas_call(
        _tc_f_body,
        grid=grid,
        in_specs=[
            pl.BlockSpec((NC, BR, OUT_FT), lambda i: (0, i, 0)),
            pl.BlockSpec((BR, OUT_FT), lambda i: (i, 0)),
            pl.BlockSpec((BR, 1), lambda i: (i, 0)),
            pl.BlockSpec((1, OUT_FT), lambda i: (0, 0)),
            pl.BlockSpec((1, 1), lambda i: (0, 0)),
        ],
        out_specs=pl.BlockSpec((BR, OUT_FT), lambda i: (i, 0)),
        out_shape=jax.ShapeDtypeStruct((N, OUT_FT), jnp.float32),
    )(s2[:, :N, :], g2, dinv, b2.reshape(1, OUT_FT), a2)
    return out


# speedup vs baseline: 16.2877x; 1.0677x over previous
import functools

import jax
import jax.numpy as jnp
from jax import lax
from jax.experimental import pallas as pl
from jax.experimental.pallas import tpu as pltpu
from jax.experimental.pallas import tpu_sc as plsc

NC = 2
NS = 16
CH = 112
NB = 3


def _mesh():
    return plsc.VectorSubcoreMesh(core_axis_name="c", subcore_axis_name="s")


def _make_deg(NP, nch):
    ZR = NP // NS

    @functools.partial(
        pl.kernel,
        out_type=jax.ShapeDtypeStruct((NC * NP,), jnp.float32),
        mesh=_mesh(),
        scratch_types=[
            pltpu.VMEM((nch, 1, CH), jnp.int32),
            pltpu.VMEM((CH,), jnp.float32),
            pltpu.VMEM((ZR,), jnp.float32),
            pltpu.VMEM_SHARED((NP,), jnp.float32),
        ],
    )
    def deg_kernel(dst_hbm, out_hbm, dst_v, ones_v, zer_v, deg_sh):
        cid = lax.axis_index("c")
        sid = lax.axis_index("s")
        wid = sid * NC + cid
        for k in range(CH // 16):
            ones_v[pl.ds(k * 16, 16)] = jnp.ones((16,), jnp.float32)
        for k in range(ZR // 16):
            zer_v[pl.ds(k * 16, 16)] = jnp.zeros((16,), jnp.float32)

        pltpu.sync_copy(zer_v, deg_sh.at[pl.ds(sid * ZR, ZR)])
        pltpu.sync_copy(dst_hbm.at[wid], dst_v)
        plsc.subcore_barrier()

        def chunk(c, carry):
            pltpu.sync_copy(ones_v, deg_sh.at[dst_v.at[c, 0]], add=True)
            return carry

        lax.fori_loop(0, nch, chunk, 0)
        plsc.subcore_barrier()
        pltpu.sync_copy(deg_sh.at[pl.ds(sid * ZR, ZR)],
                        out_hbm.at[pl.ds(cid * NP + sid * ZR, ZR)])

    return deg_kernel


def _make_scatter(NP, F, nch, col_split):
    ZR = NP // NS
    assert nch % NB == 0 and ZR % 64 == 0

    @functools.partial(
        pl.kernel,
        out_type=jax.ShapeDtypeStruct((NC, NP, F), jnp.float32),
        mesh=_mesh(),
        scratch_types=[
            pltpu.VMEM((NB, 1, CH), jnp.int32),
            pltpu.VMEM((NB, 1, CH), jnp.int32),
            pltpu.VMEM((NB, CH, F), jnp.float32),
            pltpu.VMEM_SHARED((NP, F), jnp.float32),
            [pltpu.SemaphoreType.DMA] * NB,
            [pltpu.SemaphoreType.DMA] * NB,
        ],
    )
    def scat_kernel(src_hbm, dst_hbm, g_hbm, out_hbm,
                    src_v, dst_v, rows_v, acc_sh, sems_g, sems_s):
        cid = lax.axis_index("c")
        sid = lax.axis_index("s")

        def load_idx(cc, b):
            if col_split:
                pltpu.sync_copy(src_hbm.at[cid, sid, cc], src_v.at[b])
                pltpu.sync_copy(dst_hbm.at[sid, cc], dst_v.at[b])
            else:
                wid = sid * NC + cid
                pltpu.sync_copy(src_hbm.at[wid, cc], src_v.at[b])
                pltpu.sync_copy(dst_hbm.at[wid, cc], dst_v.at[b])

        def fire_gather(b):
            pltpu.async_copy(g_hbm.at[src_v.at[b, 0]], rows_v.at[b],
                             sems_g[b])

        def wait_gather(b):
            pltpu.make_async_copy(g_hbm.at[src_v.at[b, 0]], rows_v.at[b],
                                  sems_g[b]).wait()

        def fire_scatter(b):
            pltpu.async_copy(rows_v.at[b], acc_sh.at[dst_v.at[b, 0]],
                             sems_s[b], add=True)

        def wait_scatter(b):
            pltpu.make_async_copy(rows_v.at[b], acc_sh.at[dst_v.at[b, 0]],
                                  sems_s[b]).wait()

        def zrow(r, carry):
            for k in range(F // 16):
                rows_v[0, r, pl.ds(k * 16, 16)] = jnp.zeros((16,), jnp.float32)
            return carry

        lax.fori_loop(0, 64, zrow, 0)
        for z in range(ZR // 64):
            pltpu.sync_copy(rows_v.at[0, pl.ds(0, 64)],
                            acc_sh.at[pl.ds(sid * ZR + z * 64, 64)])

        NP_dummy = NP - 8
        for b in range(NB):
            for k in range(CH // 16):
                dst_v[b, 0, pl.ds(k * 16, 16)] = jnp.full(
                    (16,), NP_dummy, jnp.int32)
        for b in range(NB):
            fire_scatter(b)
        wait_scatter(0)
        load_idx(0, 0)
        fire_gather(0)
        plsc.subcore_barrier()

        def visit(cc, b, bnxt, last):
            if not last:
                wait_scatter(bnxt)
                load_idx(cc + 1, bnxt)
                fire_gather(bnxt)
            wait_gather(b)
            fire_scatter(b)

        def group(i, carry):
            for b in range(NB):
                cc = i * NB + b
                bnxt = (b + 1) % NB

                @pl.when(cc + 1 < nch)
                def _():
                    visit(cc, b, bnxt, False)

                @pl.when(cc + 1 >= nch)
                def _():
                    wait_gather(b)
                    fire_scatter(b)
            return carry

        lax.fori_loop(0, nch // NB, group, 0)
        for b in range(NB):
            wait_scatter(b)
        plsc.subcore_barrier()
        pltpu.sync_copy(acc_sh.at[pl.ds(sid * ZR, ZR)],
                        out_hbm.at[cid, pl.ds(sid * ZR, ZR)])

    return scat_kernel


def _prelu_tc(t, a):
    return jnp.where(t >= 0, t, a * t)


def _tc_a_body(x_ref, w_ref, degp_ref, g_ref, dinv_ref):
    deg = degp_ref[:, 0] + degp_ref[:, 1] + 1.0
    dinv = lax.rsqrt(deg)[:, None]
    h = jnp.dot(x_ref[...], w_ref[...], preferred_element_type=jnp.float32)
    g = h * dinv
    F = h.shape[1] // 2
    g_ref[0] = g[:, :F]
    g_ref[1] = g[:, F:]
    dinv_ref[...] = dinv


def _tc_d_body(s1_ref, g1_ref, dinv_ref, b1_ref, w2_ref, a_ref, g2_ref):
    a = a_ref[0, 0]
    dinv = dinv_ref[...]
    F = s1_ref.shape[2]
    t0 = _prelu_tc(dinv * (s1_ref[0] + g1_ref[0]) + b1_ref[0, :F], a)
    t1 = _prelu_tc(dinv * (s1_ref[1] + g1_ref[1]) + b1_ref[0, F:], a)
    h2 = (jnp.dot(t0, w2_ref[:F], preferred_element_type=jnp.float32)
          + jnp.dot(t1, w2_ref[F:], preferred_element_type=jnp.float32))
    g2_ref[...] = h2 * dinv


def _tc_f_body(s2_ref, g2_ref, dinv_ref, b2_ref, a_ref, out_ref):
    a = a_ref[0, 0]
    dinv = dinv_ref[...]
    t = dinv * (s2_ref[0] + s2_ref[1] + g2_ref[...]) + b2_ref[0, :]
    out_ref[...] = _prelu_tc(t, a)


def kernel(x, edge_index, W1, b1, W2, b2, a):
    N, IN_FT = x.shape
    HID = W1.shape[1]
    OUT_FT = W2.shape[1]
    E = edge_index.shape[1]
    F1 = HID // 2
    NW = NC * NS

    NP = ((N + NS * 64 - 1) // (NS * 64)) * (NS * 64)
    if NP == N:
        NP += NS * 64
    EP = ((E + NW * CH * NB - 1) // (NW * CH * NB)) * (NW * CH * NB)
    nch1 = EP // (NS * CH)
    nch2 = EP // (NW * CH)

    src = edge_index[0]
    dst = edge_index[1]
    pad = EP - E
    src_p = jnp.concatenate([src, jnp.zeros((pad,), jnp.int32)])
    dst_p = jnp.concatenate([dst, jnp.full((pad,), N, jnp.int32)])
    src16 = src_p.reshape(NS, nch1, 1, CH)
    src16b = jnp.stack([src16, src16 + N])
    dst16 = dst_p.reshape(NS, nch1, 1, CH)
    src32 = src_p.reshape(NW, nch2, 1, CH)
    coff = (jnp.arange(NW, dtype=jnp.int32) % NC)[:, None, None, None] * N
    src32b = src32 + coff
    dst32 = dst_p.reshape(NW, nch2, 1, CH)

    degp = _make_deg(NP, nch2)(dst32).reshape(NC, NP)

    BR = 1000
    assert N % BR == 0
    grid = (N // BR,)
    g1, dinv = pl.pallas_call(
        _tc_a_body,
        grid=grid,
        in_specs=[
            pl.BlockSpec((BR, IN_FT), lambda i: (i, 0)),
            pl.BlockSpec((IN_FT, HID), lambda i: (0, 0)),
            pl.BlockSpec((BR, NC), lambda i: (i, 0)),
        ],
        out_specs=[
            pl.BlockSpec((NC, BR, F1), lambda i: (0, i, 0)),
            pl.BlockSpec((BR, 1), lambda i: (i, 0)),
        ],
        out_shape=[
            jax.ShapeDtypeStruct((NC, N, F1), jnp.float32),
            jax.ShapeDtypeStruct((N, 1), jnp.float32),
        ],
    )(x, W1, degp[:, :N].T)

    g1_tab = g1.reshape(NC * N, F1)
    s1 = _make_scatter(NP, F1, nch1, True)(src16b, dst16, g1_tab)

    a2 = a.reshape(1, 1)
    g2 = pl.pallas_call(
        _tc_d_body,
        grid=grid,
        in_specs=[
            pl.BlockSpec((NC, BR, F1), lambda i: (0, i, 0)),
            pl.BlockSpec((NC, BR, F1), lambda i: (0, i, 0)),
            pl.BlockSpec((BR, 1), lambda i: (i, 0)),
            pl.BlockSpec((1, HID), lambda i: (0, 0)),
            pl.BlockSpec((HID, OUT_FT), lambda i: (0, 0)),
            pl.BlockSpec((1, 1), lambda i: (0, 0)),
        ],
        out_specs=pl.BlockSpec((BR, OUT_FT), lambda i: (i, 0)),
        out_shape=jax.ShapeDtypeStruct((N, OUT_FT), jnp.float32),
    )(s1[:, :N], g1, dinv, b1.reshape(1, HID), W2, a2)

    g2_tab = jnp.concatenate([g2, g2], axis=0)
    s2 = _make_scatter(NP, OUT_FT, nch2, False)(src32b, dst32, g2_tab)

    out = pl.pallas_call(
        _tc_f_body,
        grid=grid,
        in_specs=[
            pl.BlockSpec((NC, BR, OUT_FT), lambda i: (0, i, 0)),
            pl.BlockSpec((BR, OUT_FT), lambda i: (i, 0)),
            pl.BlockSpec((BR, 1), lambda i: (i, 0)),
            pl.BlockSpec((1, OUT_FT), lambda i: (0, 0)),
            pl.BlockSpec((1, 1), lambda i: (0, 0)),
        ],
        out_specs=pl.BlockSpec((BR, OUT_FT), lambda i: (i, 0)),
        out_shape=jax.ShapeDtypeStruct((N, OUT_FT), jnp.float32),
    )(s2[:, :N, :], g2, dinv, b2.reshape(1, OUT_FT), a2)
    return out
